# Initial kernel scaffold; baseline (speedup 1.0000x reference)
import jax
import jax.numpy as jnp
from jax.experimental import pallas as pl


def kernel(inp_xyz, inp_vals, query_xyz, conv_w0, conv_b0, bn_g0, bn_b0, conv_w1, conv_b1, bn_g1, bn_b1, conv_w2, conv_b2, bn_g2, bn_b2, lin_w, lin_b):
    raise NotImplementedError("write your pallas kernel here")


# speedup vs baseline: 7.9846x; 7.9846x over previous
import functools
import jax
import jax.numpy as jnp
from jax import lax
from jax.experimental import pallas as pl
from jax.experimental.pallas import tpu as pltpu
from jax.experimental.pallas import tpu_sc as plsc

B, N, S, CHIN, CHOUT, K = 2, 10000, 4096, 32, 64, 32
EPS = 1e-5
NPAD = 10240
TS = 256
TM = 128
RM = TM * K
M = B * S * K
NEG = 2 ** 30


def _knn_body(pt_ref, q_ref, o_ref):
    b = pl.program_id(0)
    pt = pt_ref[0]
    q = q_ref[0]
    pl2 = jnp.sum(pt * pt, axis=0, keepdims=True)
    q2 = jnp.sum(q * q, axis=1, keepdims=True)
    mm = jnp.dot(q.astype(jnp.bfloat16), pt.astype(jnp.bfloat16),
                 preferred_element_type=jnp.float32)
    d = (-2.0 * mm + q2) + pl2
    ii = lax.broadcasted_iota(jnp.int32, (TS, NPAD), 1)
    d = jnp.where(ii < N, d, jnp.inf)
    idxs = []
    for _ in range(K):
        m = jnp.min(d, axis=1, keepdims=True)
        cand = jnp.where(d == m, ii, NEG)
        sel = jnp.min(cand, axis=1)
        idxs.append(sel)
        d = jnp.where(ii == sel[:, None], jnp.inf, d)
    o_ref[0] = jnp.stack(idxs, axis=1) + b * N


def _knn(inp_xyz, query_xyz):
    ptp = jnp.zeros((B, 3, NPAD), jnp.float32)
    ptp = ptp.at[:, :, :N].set(jnp.transpose(inp_xyz, (0, 2, 1)))
    return pl.pallas_call(
        _knn_body,
        grid=(B, S // TS),
        in_specs=[
            pl.BlockSpec((1, 3, NPAD), lambda b, s: (b, 0, 0)),
            pl.BlockSpec((1, TS, 3), lambda b, s: (b, s, 0)),
        ],
        out_specs=pl.BlockSpec((1, TS, K), lambda b, s: (b, s, 0)),
        out_shape=jax.ShapeDtypeStruct((B, S, K), jnp.int32),
    )(ptp, query_xyz)


def _sc_gather(table_x, table_v, idx_flat):
    info = plsc.get_sparse_core_info()
    nw = info.num_cores * info.num_subcores
    per_w = M // nw
    ch = 128
    nchunk = per_w // ch
    mesh = plsc.VectorSubcoreMesh(core_axis_name="c", subcore_axis_name="s")

    @functools.partial(
        pl.kernel, mesh=mesh,
        compiler_params=pltpu.CompilerParams(use_tc_tiling_on_sc=False),
        out_type=[
            jax.ShapeDtypeStruct((M, 16), jnp.float32),
            jax.ShapeDtypeStruct((M, CHIN), jnp.float32),
        ],
        scratch_types=[
            pltpu.VMEM((ch,), jnp.int32),
            pltpu.VMEM((ch, 16), jnp.float32),
            pltpu.VMEM((ch, CHIN), jnp.float32),
            pltpu.SemaphoreType.DMA,
            pltpu.SemaphoreType.DMA,
        ],
    )
    def k(tx_hbm, tv_hbm, idx_hbm, ox_hbm, ov_hbm, idx_v, x_v, v_v, s1, s2):
        wid = lax.axis_index("s") * info.num_cores + lax.axis_index("c")

        def body(c, _):
            base = wid * per_w + c * ch
            pltpu.sync_copy(idx_hbm.at[pl.ds(base, ch)], idx_v)
            cp1 = pltpu.async_copy(tx_hbm.at[idx_v], x_v, s1)
            cp2 = pltpu.async_copy(tv_hbm.at[idx_v], v_v, s2)
            cp1.wait()
            cp2.wait()
            pltpu.sync_copy(x_v, ox_hbm.at[pl.ds(base, ch)])
            pltpu.sync_copy(v_v, ov_hbm.at[pl.ds(base, ch)])
            return 0

        lax.fori_loop(0, nchunk, body, 0)

    return k(table_x, table_v, idx_flat)


def _expand():
    r = lax.broadcasted_iota(jnp.int32, (RM, TM), 0)
    t = lax.broadcasted_iota(jnp.int32, (RM, TM), 1)
    return (r // K == t).astype(jnp.float32)


def _h0(gx, q, w0a):
    e = _expand()
    qe = jnp.dot(e, q, precision=lax.Precision.HIGHEST,
                 preferred_element_type=jnp.float32)
    delta = gx - qe
    return jax.nn.relu(jnp.dot(delta, w0a, preferred_element_type=jnp.float32))


def _acc_stats(o_ref, z):
    @pl.when(pl.program_id(0) == 0)
    def _():
        o_ref[...] = jnp.zeros_like(o_ref)
    o_ref[0:1, :z.shape[1]] += jnp.sum(z, axis=0, keepdims=True)
    o_ref[1:2, :z.shape[1]] += jnp.sum(z * z, axis=0, keepdims=True)


def _stats_call(body, ops, extra):
    return pl.pallas_call(
        body,
        grid=(B * S // TM,),
        in_specs=[
            pl.BlockSpec((RM, 16), lambda g: (g, 0)),
            pl.BlockSpec((TM, 16), lambda g: (g, 0)),
        ] + extra,
        out_specs=pl.BlockSpec((8, 32), lambda g: (0, 0)),
        out_shape=jax.ShapeDtypeStruct((8, 32), jnp.float32),
    )(*ops)


def _bn_fold(s, ss, b, g, beta):
    zmean = s / M
    var = ss / M - zmean * zmean
    scale = g / jnp.sqrt(var + EPS)
    btot = -zmean * scale + beta
    return scale, btot


def kernel(inp_xyz, inp_vals, query_xyz, conv_w0, conv_b0, bn_g0, bn_b0,
           conv_w1, conv_b1, bn_g1, bn_b1, conv_w2, conv_b2, bn_g2, bn_b2,
           lin_w, lin_b):
    idx = _knn(inp_xyz, query_xyz)
    table_x = jnp.zeros((B * N, 16), jnp.float32).at[:, :3].set(
        inp_xyz.reshape(B * N, 3))
    table_v = inp_vals.reshape(B * N, CHIN)
    gx, gv = _sc_gather(table_x, table_v, idx.reshape(-1))

    qf = jnp.zeros((B * S, 16), jnp.float32).at[:, :3].set(
        query_xyz.reshape(B * S, 3)).at[:, 3].set(-1.0)

    w0t = jnp.zeros((16, 32), jnp.float32).at[:3, :].set(conv_w0.T)

    def s0_body(gx_ref, q_ref, w_ref, o_ref):
        e = _expand()
        qe = jnp.dot(e, q_ref[...], precision=lax.Precision.HIGHEST,
                     preferred_element_type=jnp.float32)
        z = jnp.dot(gx_ref[...] - qe, w_ref[...],
                    preferred_element_type=jnp.float32)
        _acc_stats(o_ref, z)

    st0 = _stats_call(s0_body, [gx, qf, w0t],
                      [pl.BlockSpec((16, 32), lambda g: (0, 0))])
    sc0, bt0 = _bn_fold(st0[0], st0[1], conv_b0, bn_g0, bn_b0)
    w0a = jnp.zeros((16, 32), jnp.float32).at[:3, :].set(
        (conv_w0 * sc0[:, None]).T).at[3, :].set(bt0)

    def s1_body(gx_ref, q_ref, w0_ref, w1_ref, o_ref):
        h0 = _h0(gx_ref[...], q_ref[...], w0_ref[...])
        z = jnp.dot(h0, w1_ref[...], preferred_element_type=jnp.float32)
        _acc_stats(o_ref, z)

    st1 = _stats_call(s1_body, [gx, qf, w0a, conv_w1.T],
                      [pl.BlockSpec((16, 32), lambda g: (0, 0)),
                       pl.BlockSpec((32, 32), lambda g: (0, 0))])
    sc1, bt1 = _bn_fold(st1[0], st1[1], conv_b1, bn_g1, bn_b1)
    w1s = (conv_w1 * sc1[:, None]).T

    def s2_body(gx_ref, q_ref, w0_ref, w1_ref, pk_ref, w2_ref, o_ref):
        h0 = _h0(gx_ref[...], q_ref[...], w0_ref[...])
        h1 = jax.nn.relu(jnp.dot(h0, w1_ref[...],
                                 preferred_element_type=jnp.float32)
                         + pk_ref[0:1, :32])
        z = jnp.dot(h1, w2_ref[...], preferred_element_type=jnp.float32)
        _acc_stats(o_ref, z)

    pk1 = jnp.zeros((8, 64), jnp.float32).at[0, :32].set(bt1)
    st2 = _stats_call(s2_body, [gx, qf, w0a, w1s, pk1, conv_w2.T],
                      [pl.BlockSpec((16, 32), lambda g: (0, 0)),
                       pl.BlockSpec((32, 32), lambda g: (0, 0)),
                       pl.BlockSpec((8, 64), lambda g: (0, 0)),
                       pl.BlockSpec((32, 16), lambda g: (0, 0))])
    sc2, bt2 = _bn_fold(st2[0, :16], st2[1, :16], conv_b2, bn_g2, bn_b2)
    w2s = (conv_w2 * sc2[:, None]).T

    wr = (lin_w / float(K)).reshape(CHOUT, CHIN, 16).transpose(2, 1, 0)
    pk = jnp.zeros((8, 64), jnp.float32).at[0, :32].set(bt1) \
        .at[1, :16].set(bt2).at[2, :].set(lin_b / float(K))

    def fin_body(gx_ref, q_ref, gv_ref, w0_ref, w1_ref, w2_ref, pk_ref,
                 wr_ref, o_ref):
        e = _expand()
        h0 = _h0(gx_ref[...], q_ref[...], w0_ref[...])
        h1 = jax.nn.relu(jnp.dot(h0, w1_ref[...],
                                 preferred_element_type=jnp.float32)
                         + pk_ref[0:1, :32])
        h2 = jax.nn.relu(jnp.dot(h1, w2_ref[...],
                                 preferred_element_type=jnp.float32)
                         + pk_ref[1:2, :16])
        vals = gv_ref[...]
        et = e.T
        acc = jnp.broadcast_to(pk_ref[2:3, :], (TM, CHOUT)) * 1.0
        for o in range(16):
            po = jnp.dot(et, vals * h2[:, o:o + 1],
                         preferred_element_type=jnp.float32)
            acc = acc + jnp.dot(po, wr_ref[o],
                                preferred_element_type=jnp.float32)
        o_ref[...] = acc

    out = pl.pallas_call(
        fin_body,
        grid=(B * S // TM,),
        in_specs=[
            pl.BlockSpec((RM, 16), lambda g: (g, 0)),
            pl.BlockSpec((TM, 16), lambda g: (g, 0)),
            pl.BlockSpec((RM, CHIN), lambda g: (g, 0)),
            pl.BlockSpec((16, 32), lambda g: (0, 0)),
            pl.BlockSpec((32, 32), lambda g: (0, 0)),
            pl.BlockSpec((32, 16), lambda g: (0, 0)),
            pl.BlockSpec((8, 64), lambda g: (0, 0)),
            pl.BlockSpec((16, 32, 64), lambda g: (0, 0, 0)),
        ],
        out_specs=pl.BlockSpec((TM, CHOUT), lambda g: (g, 0)),
        out_shape=jax.ShapeDtypeStruct((B * S, CHOUT), jnp.float32),
    )(gx, qf, gv, w0a, w1s, w2s, pk, wr)
    return out.reshape(B, S, CHOUT)


# speedup vs baseline: 7.9893x; 1.0006x over previous
import functools
import jax
import jax.numpy as jnp
from jax import lax
from jax.experimental import pallas as pl
from jax.experimental.pallas import tpu as pltpu
from jax.experimental.pallas import tpu_sc as plsc

B, N, S, CHIN, CHOUT, K = 2, 10000, 4096, 32, 64, 32
EPS = 1e-5
NPAD = 10240
TS = 256
TM = 128
RM = TM * K
M = B * S * K
NEG = 2 ** 30


def _knn_body(pt_ref, q_ref, o_ref):
    b = pl.program_id(0)
    pt = pt_ref[0]
    q = q_ref[0]
    pl2 = jnp.sum(pt * pt, axis=0, keepdims=True)
    q2 = jnp.sum(q * q, axis=1, keepdims=True)
    mm = jnp.dot(q.astype(jnp.bfloat16), pt.astype(jnp.bfloat16),
                 preferred_element_type=jnp.float32)
    d = (-2.0 * mm + q2) + pl2
    ii = lax.broadcasted_iota(jnp.int32, (TS, NPAD), 1)
    d = jnp.where(ii < N, d, jnp.inf)
    idxs = []
    sel = None
    for _ in range(K):
        if sel is not None:
            d = jnp.where(ii == sel[:, None], jnp.inf, d)
        m = jnp.min(d, axis=1, keepdims=True)
        cand = jnp.where(d == m, ii, NEG)
        sel = jnp.min(cand, axis=1)
        idxs.append(sel)
    o_ref[0] = jnp.stack(idxs, axis=1) + b * N


def _knn(inp_xyz, query_xyz):
    ptp = jnp.zeros((B, 3, NPAD), jnp.float32)
    ptp = ptp.at[:, :, :N].set(jnp.transpose(inp_xyz, (0, 2, 1)))
    return pl.pallas_call(
        _knn_body,
        grid=(B, S // TS),
        in_specs=[
            pl.BlockSpec((1, 3, NPAD), lambda b, s: (b, 0, 0)),
            pl.BlockSpec((1, TS, 3), lambda b, s: (b, s, 0)),
        ],
        out_specs=pl.BlockSpec((1, TS, K), lambda b, s: (b, s, 0)),
        out_shape=jax.ShapeDtypeStruct((B, S, K), jnp.int32),
    )(ptp, query_xyz)


def _sc_gather(table_x, table_v, idx_flat):
    info = plsc.get_sparse_core_info()
    nw = info.num_cores * info.num_subcores
    per_w = M // nw
    ch = 128
    nchunk = per_w // ch
    mesh = plsc.VectorSubcoreMesh(core_axis_name="c", subcore_axis_name="s")

    @functools.partial(
        pl.kernel, mesh=mesh,
        compiler_params=pltpu.CompilerParams(use_tc_tiling_on_sc=False),
        out_type=[
            jax.ShapeDtypeStruct((M, 16), jnp.float32),
            jax.ShapeDtypeStruct((M, CHIN), jnp.float32),
        ],
        scratch_types=[
            pltpu.VMEM((ch,), jnp.int32),
            pltpu.VMEM((ch, 16), jnp.float32),
            pltpu.VMEM((ch, CHIN), jnp.float32),
            pltpu.SemaphoreType.DMA,
            pltpu.SemaphoreType.DMA,
        ],
    )
    def k(tx_hbm, tv_hbm, idx_hbm, ox_hbm, ov_hbm, idx_v, x_v, v_v, s1, s2):
        wid = lax.axis_index("s") * info.num_cores + lax.axis_index("c")

        def body(c, _):
            base = wid * per_w + c * ch
            pltpu.sync_copy(idx_hbm.at[pl.ds(base, ch)], idx_v)
            cp1 = pltpu.async_copy(tx_hbm.at[idx_v], x_v, s1)
            cp2 = pltpu.async_copy(tv_hbm.at[idx_v], v_v, s2)
            cp1.wait()
            cp2.wait()
            pltpu.sync_copy(x_v, ox_hbm.at[pl.ds(base, ch)])
            pltpu.sync_copy(v_v, ov_hbm.at[pl.ds(base, ch)])
            return 0

        lax.fori_loop(0, nchunk, body, 0)

    return k(table_x, table_v, idx_flat)


def _expand():
    r = lax.broadcasted_iota(jnp.int32, (RM, TM), 0)
    t = lax.broadcasted_iota(jnp.int32, (RM, TM), 1)
    return (r // K == t).astype(jnp.float32)


def _h0(gx, q, w0a):
    e = _expand()
    qe = jnp.dot(e, q, precision=lax.Precision.HIGHEST,
                 preferred_element_type=jnp.float32)
    delta = gx - qe
    return jax.nn.relu(jnp.dot(delta, w0a, preferred_element_type=jnp.float32))


def _acc_stats(o_ref, z):
    @pl.when(pl.program_id(0) == 0)
    def _():
        o_ref[...] = jnp.zeros_like(o_ref)
    o_ref[0:1, :z.shape[1]] += jnp.sum(z, axis=0, keepdims=True)
    o_ref[1:2, :z.shape[1]] += jnp.sum(z * z, axis=0, keepdims=True)


def _stats_call(body, ops, extra):
    return pl.pallas_call(
        body,
        grid=(B * S // TM,),
        in_specs=[
            pl.BlockSpec((RM, 16), lambda g: (g, 0)),
            pl.BlockSpec((TM, 16), lambda g: (g, 0)),
        ] + extra,
        out_specs=pl.BlockSpec((8, 32), lambda g: (0, 0)),
        out_shape=jax.ShapeDtypeStruct((8, 32), jnp.float32),
    )(*ops)


def _bn_fold(s, ss, b, g, beta):
    zmean = s / M
    var = ss / M - zmean * zmean
    scale = g / jnp.sqrt(var + EPS)
    btot = -zmean * scale + beta
    return scale, btot


def kernel(inp_xyz, inp_vals, query_xyz, conv_w0, conv_b0, bn_g0, bn_b0,
           conv_w1, conv_b1, bn_g1, bn_b1, conv_w2, conv_b2, bn_g2, bn_b2,
           lin_w, lin_b):
    idx = _knn(inp_xyz, query_xyz)
    table_x = jnp.zeros((B * N, 16), jnp.float32).at[:, :3].set(
        inp_xyz.reshape(B * N, 3))
    table_v = inp_vals.reshape(B * N, CHIN)
    gx, gv = _sc_gather(table_x, table_v, idx.reshape(-1))

    qf = jnp.zeros((B * S, 16), jnp.float32).at[:, :3].set(
        query_xyz.reshape(B * S, 3)).at[:, 3].set(-1.0)

    w0t = jnp.zeros((16, 32), jnp.float32).at[:3, :].set(conv_w0.T)

    def s0_body(gx_ref, q_ref, w_ref, o_ref):
        e = _expand()
        qe = jnp.dot(e, q_ref[...], precision=lax.Precision.HIGHEST,
                     preferred_element_type=jnp.float32)
        z = jnp.dot(gx_ref[...] - qe, w_ref[...],
                    preferred_element_type=jnp.float32)
        _acc_stats(o_ref, z)

    st0 = _stats_call(s0_body, [gx, qf, w0t],
                      [pl.BlockSpec((16, 32), lambda g: (0, 0))])
    sc0, bt0 = _bn_fold(st0[0], st0[1], conv_b0, bn_g0, bn_b0)
    w0a = jnp.zeros((16, 32), jnp.float32).at[:3, :].set(
        (conv_w0 * sc0[:, None]).T).at[3, :].set(bt0)

    def s1_body(gx_ref, q_ref, w0_ref, w1_ref, o_ref):
        h0 = _h0(gx_ref[...], q_ref[...], w0_ref[...])
        z = jnp.dot(h0, w1_ref[...], preferred_element_type=jnp.float32)
        _acc_stats(o_ref, z)

    st1 = _stats_call(s1_body, [gx, qf, w0a, conv_w1.T],
                      [pl.BlockSpec((16, 32), lambda g: (0, 0)),
                       pl.BlockSpec((32, 32), lambda g: (0, 0))])
    sc1, bt1 = _bn_fold(st1[0], st1[1], conv_b1, bn_g1, bn_b1)
    w1s = (conv_w1 * sc1[:, None]).T

    def s2_body(gx_ref, q_ref, w0_ref, w1_ref, pk_ref, w2_ref, o_ref):
        h0 = _h0(gx_ref[...], q_ref[...], w0_ref[...])
        h1 = jax.nn.relu(jnp.dot(h0, w1_ref[...],
                                 preferred_element_type=jnp.float32)
                         + pk_ref[0:1, :32])
        z = jnp.dot(h1, w2_ref[...], preferred_element_type=jnp.float32)
        _acc_stats(o_ref, z)

    pk1 = jnp.zeros((8, 64), jnp.float32).at[0, :32].set(bt1)
    st2 = _stats_call(s2_body, [gx, qf, w0a, w1s, pk1, conv_w2.T],
                      [pl.BlockSpec((16, 32), lambda g: (0, 0)),
                       pl.BlockSpec((32, 32), lambda g: (0, 0)),
                       pl.BlockSpec((8, 64), lambda g: (0, 0)),
                       pl.BlockSpec((32, 16), lambda g: (0, 0))])
    sc2, bt2 = _bn_fold(st2[0, :16], st2[1, :16], conv_b2, bn_g2, bn_b2)
    w2s = (conv_w2 * sc2[:, None]).T

    wr = (lin_w / float(K)).reshape(CHOUT, CHIN, 16).transpose(2, 1, 0)
    pk = jnp.zeros((8, 64), jnp.float32).at[0, :32].set(bt1) \
        .at[1, :16].set(bt2).at[2, :].set(lin_b / float(K))

    def fin_body(gx_ref, q_ref, gv_ref, w0_ref, w1_ref, w2_ref, pk_ref,
                 wr_ref, o_ref):
        e = _expand()
        h0 = _h0(gx_ref[...], q_ref[...], w0_ref[...])
        h1 = jax.nn.relu(jnp.dot(h0, w1_ref[...],
                                 preferred_element_type=jnp.float32)
                         + pk_ref[0:1, :32])
        h2 = jax.nn.relu(jnp.dot(h1, w2_ref[...],
                                 preferred_element_type=jnp.float32)
                         + pk_ref[1:2, :16])
        vals = gv_ref[...]
        et = e.T
        acc = jnp.broadcast_to(pk_ref[2:3, :], (TM, CHOUT)) * 1.0
        for o in range(16):
            po = jnp.dot(et, vals * h2[:, o:o + 1],
                         preferred_element_type=jnp.float32)
            acc = acc + jnp.dot(po, wr_ref[o],
                                preferred_element_type=jnp.float32)
        o_ref[...] = acc

    out = pl.pallas_call(
        fin_body,
        grid=(B * S // TM,),
        in_specs=[
            pl.BlockSpec((RM, 16), lambda g: (g, 0)),
            pl.BlockSpec((TM, 16), lambda g: (g, 0)),
            pl.BlockSpec((RM, CHIN), lambda g: (g, 0)),
            pl.BlockSpec((16, 32), lambda g: (0, 0)),
            pl.BlockSpec((32, 32), lambda g: (0, 0)),
            pl.BlockSpec((32, 16), lambda g: (0, 0)),
            pl.BlockSpec((8, 64), lambda g: (0, 0)),
            pl.BlockSpec((16, 32, 64), lambda g: (0, 0, 0)),
        ],
        out_specs=pl.BlockSpec((TM, CHOUT), lambda g: (g, 0)),
        out_shape=jax.ShapeDtypeStruct((B * S, CHOUT), jnp.float32),
    )(gx, qf, gv, w0a, w1s, w2s, pk, wr)
    return out.reshape(B, S, CHOUT)
